# baseline (speedup 1.0000x reference)
import functools

import numpy as np
import jax
import jax.numpy as jnp
from jax import lax
from jax.experimental import pallas as pl
from jax.experimental.pallas import tpu as pltpu
from jax.experimental.pallas import tpu_sc as plsc

_L = 32
_AG = 32
_B = 4096
_N = _AG + _L
_C = _N * (_N - 1) // 2

_TRIU_NP = np.stack(np.triu_indices(_N, k=1)).astype(np.int32)


_BB = 64


def _tc_body(pos_ref, vel_ref, rel_l_ref, rel_o_ref, comm_ref,
             x_ref, ea_ref, bv_ref):
    pos = pos_ref[...]
    vel = vel_ref[...]
    rel_l = rel_l_ref[...]
    rel_o = rel_o_ref[...]
    comm = comm_ref[...]
    denom = 0.001 + vel

    land = jnp.concatenate(
        [pos + rel_l, rel_l, rel_l / denom,
         jnp.zeros((_BB, _L, 2), jnp.float32)], axis=2)
    oth = jnp.concatenate(
        [pos + rel_o, rel_o, rel_o / denom,
         jnp.ones((_BB, _AG - 1, 1), jnp.float32), comm], axis=2)
    agent = jnp.concatenate(
        [pos, vel, jnp.zeros((_BB, 1, 2), jnp.float32),
         jnp.full((_BB, 1, 1), 2.0, jnp.float32),
         jnp.zeros((_BB, 1, 1), jnp.float32)], axis=2)
    x_ref[...] = jnp.concatenate([agent, land, oth], axis=1)

    rl = rel_l[:, :2, :]
    ss = jnp.sum(rl * rl + (vel - rl) ** 2 + (rl / denom) ** 2, axis=2) + 4.0
    ea_ref[...] = jnp.sqrt(ss)

    i0 = pl.program_id(0) * _BB
    bv_ref[...] = i0 + lax.broadcasted_iota(jnp.int32, (_BB, _N), 0)


_tc_call = pl.pallas_call(
    _tc_body,
    grid=(_B // _BB,),
    in_specs=[
        pl.BlockSpec((_BB, 1, 2), lambda i: (i, 0, 0)),
        pl.BlockSpec((_BB, 1, 2), lambda i: (i, 0, 0)),
        pl.BlockSpec((_BB, _L, 2), lambda i: (i, 0, 0)),
        pl.BlockSpec((_BB, _AG - 1, 2), lambda i: (i, 0, 0)),
        pl.BlockSpec((_BB, _AG - 1, 1), lambda i: (i, 0, 0)),
    ],
    out_specs=[
        pl.BlockSpec((_BB, _N, 8), lambda i: (i, 0, 0)),
        pl.BlockSpec((_BB, 2), lambda i: (i, 0)),
        pl.BlockSpec((_BB, _N), lambda i: (i, 0)),
    ],
    out_shape=[
        jax.ShapeDtypeStruct((_B, _N, 8), jnp.float32),
        jax.ShapeDtypeStruct((_B, 2), jnp.float32),
        jax.ShapeDtypeStruct((_B, _N), jnp.int32),
    ],
)


_NW = 32
_BPW = _B // _NW
_K = 8
_NCHUNK = _BPW // _K


def _sc_ei_body(triu_hbm, ei_hbm, base_v, buf_r, buf_c):
    core = lax.axis_index("c")
    sub = lax.axis_index("s")
    wid = sub * 2 + core
    pltpu.sync_copy(triu_hbm, base_v)
    i0 = wid * _BPW

    def chunk_body(ch, _):
        start = i0 + ch * _K

        def batch_body(k, _):
            off = (start + k) * _N

            def vec_body(v, _):
                sl = pl.ds(v * 16, 16)
                buf_r[k, sl] = base_v[0, sl] + off
                buf_c[k, sl] = base_v[1, sl] + off
                return ()

            lax.fori_loop(0, _C // 16, vec_body, ())
            return ()

        lax.fori_loop(0, _K, batch_body, ())
        pltpu.sync_copy(buf_r, ei_hbm.at[0, pl.ds(start, _K)])
        pltpu.sync_copy(buf_c, ei_hbm.at[1, pl.ds(start, _K)])
        return ()

    lax.fori_loop(0, _NCHUNK, chunk_body, ())


@functools.lru_cache(maxsize=None)
def _sc_ei_call():
    return pl.kernel(
        _sc_ei_body,
        out_type=jax.ShapeDtypeStruct((2, _B, _C), jnp.int32),
        mesh=plsc.VectorSubcoreMesh(core_axis_name="c", subcore_axis_name="s"),
        scratch_types=[
            pltpu.VMEM((2, _C), jnp.int32),
            pltpu.VMEM((_K, _C), jnp.int32),
            pltpu.VMEM((_K, _C), jnp.int32),
        ],
    )


def kernel(batch_observations):
    obs = batch_observations
    vel3 = obs[:, 0:2].reshape(_B, 1, 2)
    pos3 = obs[:, 2:4].reshape(_B, 1, 2)
    rel_l = obs[:, 4:4 + 2 * _L].reshape(_B, _L, 2)
    rel_o = obs[:, 4 + 2 * _L:4 + 2 * (_L + _AG - 1)].reshape(_B, _AG - 1, 2)
    comm3 = obs[:, -(_AG - 1):].reshape(_B, _AG - 1, 1)

    x3, ea, bv = _tc_call(pos3, vel3, rel_l, rel_o, comm3)
    ei3 = _sc_ei_call()(jnp.asarray(_TRIU_NP))

    return (x3.reshape(_B * _N, 8),
            ei3.reshape(2, _B * _C),
            ea.reshape(-1),
            bv.reshape(-1))


# speedup vs baseline: 1.0001x; 1.0001x over previous
import functools

import numpy as np
import jax
import jax.numpy as jnp
from jax import lax
from jax.experimental import pallas as pl
from jax.experimental.pallas import tpu as pltpu
from jax.experimental.pallas import tpu_sc as plsc

_L = 32
_AG = 32
_B = 4096
_N = _AG + _L
_C = _N * (_N - 1) // 2

_TRIU_NP = np.stack(np.triu_indices(_N, k=1)).astype(np.int32)


_BB = 64


def _tc_body(pos_ref, vel_ref, rel_l_ref, rel_o_ref, comm_ref,
             x_ref, ea_ref, bv_ref):
    pos = pos_ref[...]
    vel = vel_ref[...]
    rel_l = rel_l_ref[...]
    rel_o = rel_o_ref[...]
    comm = comm_ref[...]
    denom = 0.001 + vel

    land = jnp.concatenate(
        [pos + rel_l, rel_l, rel_l / denom,
         jnp.zeros((_BB, _L, 2), jnp.float32)], axis=2)
    oth = jnp.concatenate(
        [pos + rel_o, rel_o, rel_o / denom,
         jnp.ones((_BB, _AG - 1, 1), jnp.float32), comm], axis=2)
    agent = jnp.concatenate(
        [pos, vel, jnp.zeros((_BB, 1, 2), jnp.float32),
         jnp.full((_BB, 1, 1), 2.0, jnp.float32),
         jnp.zeros((_BB, 1, 1), jnp.float32)], axis=2)
    x_ref[...] = jnp.concatenate([agent, land, oth], axis=1)

    rl = rel_l[:, :2, :]
    ss = jnp.sum(rl * rl + (vel - rl) ** 2 + (rl / denom) ** 2, axis=2) + 4.0
    ea_ref[...] = jnp.sqrt(ss)

    i0 = pl.program_id(0) * _BB
    bv_ref[...] = i0 + lax.broadcasted_iota(jnp.int32, (_BB, _N), 0)


_tc_call = pl.pallas_call(
    _tc_body,
    grid=(_B // _BB,),
    in_specs=[
        pl.BlockSpec((_BB, 1, 2), lambda i: (i, 0, 0)),
        pl.BlockSpec((_BB, 1, 2), lambda i: (i, 0, 0)),
        pl.BlockSpec((_BB, _L, 2), lambda i: (i, 0, 0)),
        pl.BlockSpec((_BB, _AG - 1, 2), lambda i: (i, 0, 0)),
        pl.BlockSpec((_BB, _AG - 1, 1), lambda i: (i, 0, 0)),
    ],
    out_specs=[
        pl.BlockSpec((_BB, _N, 8), lambda i: (i, 0, 0)),
        pl.BlockSpec((_BB, 2), lambda i: (i, 0)),
        pl.BlockSpec((_BB, _N), lambda i: (i, 0)),
    ],
    out_shape=[
        jax.ShapeDtypeStruct((_B, _N, 8), jnp.float32),
        jax.ShapeDtypeStruct((_B, 2), jnp.float32),
        jax.ShapeDtypeStruct((_B, _N), jnp.int32),
    ],
)


_NW = 32
_BPW = _B // _NW
_K = 8
_NCHUNK = _BPW // _K


def _sc_ei_body(triu_hbm, ei_hbm, base_v,
                b0r, b0c, b1r, b1c, sem0, sem1):
    core = lax.axis_index("c")
    sub = lax.axis_index("s")
    wid = sub * 2 + core
    pltpu.sync_copy(triu_hbm, base_v)
    i0 = wid * _BPW
    bufs = ((b0r, b0c, sem0), (b1r, b1c, sem1))

    def compute_chunk(start, br, bc):
        def vec_body(v, _):
            sl = pl.ds(v * 16, 16)
            vr = base_v[0, sl]
            vc = base_v[1, sl]
            for k in range(_K):
                off = (start + k) * _N
                br[k, sl] = vr + off
                bc[k, sl] = vc + off
            return ()

        lax.fori_loop(0, _C // 16, vec_body, ())

    def dma_out(br, bc, sem, start):
        a = pltpu.make_async_copy(br, ei_hbm.at[0, pl.ds(start, _K)], sem)
        b = pltpu.make_async_copy(bc, ei_hbm.at[1, pl.ds(start, _K)], sem)
        return a, b

    def pair_body(p, _):
        for b in range(2):
            ch = p * 2 + b
            start = i0 + ch * _K
            br, bc, sem = bufs[b]

            @pl.when(p > 0)
            def _():
                x, y = dma_out(br, bc, sem, start)
                x.wait()
                y.wait()

            compute_chunk(start, br, bc)
            x, y = dma_out(br, bc, sem, start)
            x.start()
            y.start()
        return ()

    lax.fori_loop(0, _NCHUNK // 2, pair_body, ())
    for b in range(2):
        br, bc, sem = bufs[b]
        x, y = dma_out(br, bc, sem, i0)
        x.wait()
        y.wait()


@functools.lru_cache(maxsize=None)
def _sc_ei_call():
    return pl.kernel(
        _sc_ei_body,
        out_type=jax.ShapeDtypeStruct((2, _B, _C), jnp.int32),
        mesh=plsc.VectorSubcoreMesh(core_axis_name="c", subcore_axis_name="s"),
        scratch_types=[
            pltpu.VMEM((2, _C), jnp.int32),
            pltpu.VMEM((_K, _C), jnp.int32),
            pltpu.VMEM((_K, _C), jnp.int32),
            pltpu.VMEM((_K, _C), jnp.int32),
            pltpu.VMEM((_K, _C), jnp.int32),
            pltpu.SemaphoreType.DMA,
            pltpu.SemaphoreType.DMA,
        ],
    )


def kernel(batch_observations):
    obs = batch_observations
    vel3 = obs[:, 0:2].reshape(_B, 1, 2)
    pos3 = obs[:, 2:4].reshape(_B, 1, 2)
    rel_l = obs[:, 4:4 + 2 * _L].reshape(_B, _L, 2)
    rel_o = obs[:, 4 + 2 * _L:4 + 2 * (_L + _AG - 1)].reshape(_B, _AG - 1, 2)
    comm3 = obs[:, -(_AG - 1):].reshape(_B, _AG - 1, 1)

    x3, ea, bv = _tc_call(pos3, vel3, rel_l, rel_o, comm3)
    ei3 = _sc_ei_call()(jnp.asarray(_TRIU_NP))

    return (x3.reshape(_B * _N, 8),
            ei3.reshape(2, _B * _C),
            ea.reshape(-1),
            bv.reshape(-1))


# speedup vs baseline: 1.0010x; 1.0009x over previous
import functools

import numpy as np
import jax
import jax.numpy as jnp
from jax import lax
from jax.experimental import pallas as pl
from jax.experimental.pallas import tpu as pltpu
from jax.experimental.pallas import tpu_sc as plsc

_L = 32
_AG = 32
_B = 4096
_N = _AG + _L
_C = _N * (_N - 1) // 2

_TRIU_NP = np.stack(np.triu_indices(_N, k=1)).astype(np.int32)


_BB = 64


def _tc_body(pos_ref, vel_ref, rel_l_ref, rel_o_ref, comm_ref,
             x_ref, ea_ref, bv_ref):
    pos = pos_ref[...]
    vel = vel_ref[...]
    rel_l = rel_l_ref[...]
    rel_o = rel_o_ref[...]
    comm = comm_ref[...]
    denom = 0.001 + vel

    land = jnp.concatenate(
        [pos + rel_l, rel_l, rel_l / denom,
         jnp.zeros((_BB, _L, 2), jnp.float32)], axis=2)
    oth = jnp.concatenate(
        [pos + rel_o, rel_o, rel_o / denom,
         jnp.ones((_BB, _AG - 1, 1), jnp.float32), comm], axis=2)
    agent = jnp.concatenate(
        [pos, vel, jnp.zeros((_BB, 1, 2), jnp.float32),
         jnp.full((_BB, 1, 1), 2.0, jnp.float32),
         jnp.zeros((_BB, 1, 1), jnp.float32)], axis=2)
    x_ref[...] = jnp.concatenate([agent, land, oth], axis=1)

    rl = rel_l[:, :2, :]
    ss = jnp.sum(rl * rl + (vel - rl) ** 2 + (rl / denom) ** 2, axis=2) + 4.0
    ea_ref[...] = jnp.sqrt(ss)

    i0 = pl.program_id(0) * _BB
    bv_ref[...] = i0 + lax.broadcasted_iota(jnp.int32, (_BB, _N), 0)


_tc_call = pl.pallas_call(
    _tc_body,
    grid=(_B // _BB,),
    in_specs=[
        pl.BlockSpec((_BB, 1, 2), lambda i: (i, 0, 0)),
        pl.BlockSpec((_BB, 1, 2), lambda i: (i, 0, 0)),
        pl.BlockSpec((_BB, _L, 2), lambda i: (i, 0, 0)),
        pl.BlockSpec((_BB, _AG - 1, 2), lambda i: (i, 0, 0)),
        pl.BlockSpec((_BB, _AG - 1, 1), lambda i: (i, 0, 0)),
    ],
    out_specs=[
        pl.BlockSpec((_BB, _N, 8), lambda i: (i, 0, 0)),
        pl.BlockSpec((_BB, 2), lambda i: (i, 0)),
        pl.BlockSpec((_BB, _N), lambda i: (i, 0)),
    ],
    out_shape=[
        jax.ShapeDtypeStruct((_B, _N, 8), jnp.float32),
        jax.ShapeDtypeStruct((_B, 2), jnp.float32),
        jax.ShapeDtypeStruct((_B, _N), jnp.int32),
    ],
)


_NW = 32
_BPW = _B // _NW
_K = 4
_NCHUNK = _BPW // _K


def _sc_ei_body(triu_hbm, ei_hbm, base_v,
                b0r, b0c, b1r, b1c, sem0, sem1):
    core = lax.axis_index("c")
    sub = lax.axis_index("s")
    wid = sub * 2 + core
    pltpu.sync_copy(triu_hbm, base_v)
    i0 = wid * _BPW
    bufs = ((b0r, b0c, sem0), (b1r, b1c, sem1))

    def compute_chunk(start, br, bc):
        for v in range(_C // 16):
            sl = pl.ds(v * 16, 16)
            vr = base_v[0, sl]
            vc = base_v[1, sl]
            for k in range(_K):
                off = (start + k) * _N
                br[k, sl] = vr + off
                bc[k, sl] = vc + off

    def dma_out(br, bc, sem, start):
        a = pltpu.make_async_copy(br, ei_hbm.at[0, pl.ds(start, _K)], sem)
        b = pltpu.make_async_copy(bc, ei_hbm.at[1, pl.ds(start, _K)], sem)
        return a, b

    def pair_body(p, _):
        for b in range(2):
            ch = p * 2 + b
            start = i0 + ch * _K
            br, bc, sem = bufs[b]

            @pl.when(p > 0)
            def _():
                x, y = dma_out(br, bc, sem, start)
                x.wait()
                y.wait()

            compute_chunk(start, br, bc)
            x, y = dma_out(br, bc, sem, start)
            x.start()
            y.start()
        return ()

    lax.fori_loop(0, _NCHUNK // 2, pair_body, ())
    for b in range(2):
        br, bc, sem = bufs[b]
        x, y = dma_out(br, bc, sem, i0)
        x.wait()
        y.wait()


@functools.lru_cache(maxsize=None)
def _sc_ei_call():
    return pl.kernel(
        _sc_ei_body,
        out_type=jax.ShapeDtypeStruct((2, _B, _C), jnp.int32),
        mesh=plsc.VectorSubcoreMesh(core_axis_name="c", subcore_axis_name="s"),
        scratch_types=[
            pltpu.VMEM((2, _C), jnp.int32),
            pltpu.VMEM((_K, _C), jnp.int32),
            pltpu.VMEM((_K, _C), jnp.int32),
            pltpu.VMEM((_K, _C), jnp.int32),
            pltpu.VMEM((_K, _C), jnp.int32),
            pltpu.SemaphoreType.DMA,
            pltpu.SemaphoreType.DMA,
        ],
    )


def kernel(batch_observations):
    obs = batch_observations
    vel3 = obs[:, 0:2].reshape(_B, 1, 2)
    pos3 = obs[:, 2:4].reshape(_B, 1, 2)
    rel_l = obs[:, 4:4 + 2 * _L].reshape(_B, _L, 2)
    rel_o = obs[:, 4 + 2 * _L:4 + 2 * (_L + _AG - 1)].reshape(_B, _AG - 1, 2)
    comm3 = obs[:, -(_AG - 1):].reshape(_B, _AG - 1, 1)

    x3, ea, bv = _tc_call(pos3, vel3, rel_l, rel_o, comm3)
    ei3 = _sc_ei_call()(jnp.asarray(_TRIU_NP))

    return (x3.reshape(_B * _N, 8),
            ei3.reshape(2, _B * _C),
            ea.reshape(-1),
            bv.reshape(-1))


# speedup vs baseline: 3.6604x; 3.6567x over previous
import functools

import numpy as np
import jax
import jax.numpy as jnp
from jax import lax
from jax.experimental import pallas as pl
from jax.experimental.pallas import tpu as pltpu
from jax.experimental.pallas import tpu_sc as plsc

_L = 32
_AG = 32
_B = 4096
_N = _AG + _L
_C = _N * (_N - 1) // 2

_G = 8
_triu = np.stack(np.triu_indices(_N, k=1)).astype(np.int32)
_TB_NP = (np.tile(_triu, (1, _G))
          + (np.arange(_G, dtype=np.int32).repeat(_C) * _N)[None, :])


def _ei_body(tb_ref, ei_ref):
    off = pl.program_id(0) * (_G * _N)
    ei_ref[...] = tb_ref[...] + off


_ei_call = pl.pallas_call(
    _ei_body,
    grid=(_B // _G,),
    in_specs=[pl.BlockSpec((2, _G * _C), lambda i: (0, 0))],
    out_specs=pl.BlockSpec((2, _G * _C), lambda i: (0, i)),
    out_shape=jax.ShapeDtypeStruct((2, _B * _C), jnp.int32),
)


_BB = 64


def _tc_body(pos_ref, vel_ref, rel_l_ref, rel_o_ref, comm_ref,
             x_ref, ea_ref):
    pos = pos_ref[...]
    vel = vel_ref[...]
    rel_l = rel_l_ref[...]
    rel_o = rel_o_ref[...]
    comm = comm_ref[...]
    denom = 0.001 + vel

    land = jnp.concatenate(
        [pos + rel_l, rel_l, rel_l / denom,
         jnp.zeros((_BB, _L, 2), jnp.float32)], axis=2)
    oth = jnp.concatenate(
        [pos + rel_o, rel_o, rel_o / denom,
         jnp.ones((_BB, _AG - 1, 1), jnp.float32), comm], axis=2)
    agent = jnp.concatenate(
        [pos, vel, jnp.zeros((_BB, 1, 2), jnp.float32),
         jnp.full((_BB, 1, 1), 2.0, jnp.float32),
         jnp.zeros((_BB, 1, 1), jnp.float32)], axis=2)
    x_ref[...] = jnp.concatenate([agent, land, oth], axis=1)

    rl = rel_l[:, :2, :]
    ss = jnp.sum(rl * rl + (vel - rl) ** 2 + (rl / denom) ** 2, axis=2) + 4.0
    ea_ref[...] = jnp.sqrt(ss)


_tc_call = pl.pallas_call(
    _tc_body,
    grid=(_B // _BB,),
    in_specs=[
        pl.BlockSpec((_BB, 1, 2), lambda i: (i, 0, 0)),
        pl.BlockSpec((_BB, 1, 2), lambda i: (i, 0, 0)),
        pl.BlockSpec((_BB, _L, 2), lambda i: (i, 0, 0)),
        pl.BlockSpec((_BB, _AG - 1, 2), lambda i: (i, 0, 0)),
        pl.BlockSpec((_BB, _AG - 1, 1), lambda i: (i, 0, 0)),
    ],
    out_specs=[
        pl.BlockSpec((_BB, _N, 8), lambda i: (i, 0, 0)),
        pl.BlockSpec((_BB, 2), lambda i: (i, 0)),
    ],
    out_shape=[
        jax.ShapeDtypeStruct((_B, _N, 8), jnp.float32),
        jax.ShapeDtypeStruct((_B, 2), jnp.float32),
    ],
)


_NW = 32
_BPW = _B // _NW


def _sc_bv_body(bv_hbm, buf):
    core = lax.axis_index("c")
    sub = lax.axis_index("s")
    wid = sub * 2 + core
    i0 = wid * _BPW
    zero16 = lax.broadcasted_iota(jnp.int32, (16,), 0) * 0
    for k in range(_BPW):
        val = zero16 + (i0 + k)
        for v in range(_N // 16):
            buf[pl.ds(k * _N + v * 16, 16)] = val
    pltpu.sync_copy(buf, bv_hbm.at[pl.ds(i0 * _N, _BPW * _N)])


@functools.lru_cache(maxsize=None)
def _sc_bv_call():
    return pl.kernel(
        _sc_bv_body,
        out_type=jax.ShapeDtypeStruct((_B * _N,), jnp.int32),
        mesh=plsc.VectorSubcoreMesh(core_axis_name="c", subcore_axis_name="s"),
        scratch_types=[pltpu.VMEM((_BPW * _N,), jnp.int32)],
    )


def kernel(batch_observations):
    obs = batch_observations
    vel3 = obs[:, 0:2].reshape(_B, 1, 2)
    pos3 = obs[:, 2:4].reshape(_B, 1, 2)
    rel_l = obs[:, 4:4 + 2 * _L].reshape(_B, _L, 2)
    rel_o = obs[:, 4 + 2 * _L:4 + 2 * (_L + _AG - 1)].reshape(_B, _AG - 1, 2)
    comm3 = obs[:, -(_AG - 1):].reshape(_B, _AG - 1, 1)

    ei = _ei_call(jnp.asarray(_TB_NP))
    x3, ea = _tc_call(pos3, vel3, rel_l, rel_o, comm3)
    bv = _sc_bv_call()()

    return (x3.reshape(_B * _N, 8),
            ei,
            ea.reshape(-1),
            bv)


# speedup vs baseline: 5.0141x; 1.3698x over previous
import functools

import numpy as np
import jax
import jax.numpy as jnp
from jax import lax
from jax.experimental import pallas as pl
from jax.experimental.pallas import tpu as pltpu
from jax.experimental.pallas import tpu_sc as plsc

_L = 32
_AG = 32
_B = 4096
_N = _AG + _L
_C = _N * (_N - 1) // 2

_G = 64
_triu = np.stack(np.triu_indices(_N, k=1)).astype(np.int32)
_TB_NP = (np.tile(_triu, (1, _G))
          + (np.arange(_G, dtype=np.int32).repeat(_C) * _N)[None, :])


def _ei_body(tb_ref, ei_ref):
    off = pl.program_id(0) * (_G * _N)
    ei_ref[...] = tb_ref[...] + off


_ei_call = pl.pallas_call(
    _ei_body,
    grid=(_B // _G,),
    in_specs=[pl.BlockSpec((2, _G * _C), lambda i: (0, 0))],
    out_specs=pl.BlockSpec((2, _G * _C), lambda i: (0, i)),
    out_shape=jax.ShapeDtypeStruct((2, _B * _C), jnp.int32),
)


_BB = 64


def _tc_body(pos_ref, vel_ref, rel_l_ref, rel_o_ref, comm_ref,
             x_ref, ea_ref):
    pos = pos_ref[...]
    vel = vel_ref[...]
    rel_l = rel_l_ref[...]
    rel_o = rel_o_ref[...]
    comm = comm_ref[...]
    denom = 0.001 + vel

    land = jnp.concatenate(
        [pos + rel_l, rel_l, rel_l / denom,
         jnp.zeros((_BB, _L, 2), jnp.float32)], axis=2)
    oth = jnp.concatenate(
        [pos + rel_o, rel_o, rel_o / denom,
         jnp.ones((_BB, _AG - 1, 1), jnp.float32), comm], axis=2)
    agent = jnp.concatenate(
        [pos, vel, jnp.zeros((_BB, 1, 2), jnp.float32),
         jnp.full((_BB, 1, 1), 2.0, jnp.float32),
         jnp.zeros((_BB, 1, 1), jnp.float32)], axis=2)
    x_ref[...] = jnp.concatenate([agent, land, oth], axis=1)

    rl = rel_l[:, :2, :]
    ss = jnp.sum(rl * rl + (vel - rl) ** 2 + (rl / denom) ** 2, axis=2) + 4.0
    ea_ref[...] = jnp.sqrt(ss)


_tc_call = pl.pallas_call(
    _tc_body,
    grid=(_B // _BB,),
    in_specs=[
        pl.BlockSpec((_BB, 1, 2), lambda i: (i, 0, 0)),
        pl.BlockSpec((_BB, 1, 2), lambda i: (i, 0, 0)),
        pl.BlockSpec((_BB, _L, 2), lambda i: (i, 0, 0)),
        pl.BlockSpec((_BB, _AG - 1, 2), lambda i: (i, 0, 0)),
        pl.BlockSpec((_BB, _AG - 1, 1), lambda i: (i, 0, 0)),
    ],
    out_specs=[
        pl.BlockSpec((_BB, _N, 8), lambda i: (i, 0, 0)),
        pl.BlockSpec((_BB, 2), lambda i: (i, 0)),
    ],
    out_shape=[
        jax.ShapeDtypeStruct((_B, _N, 8), jnp.float32),
        jax.ShapeDtypeStruct((_B, 2), jnp.float32),
    ],
)


_NW = 32
_BPW = _B // _NW


def _sc_bv_body(bv_hbm, buf):
    core = lax.axis_index("c")
    sub = lax.axis_index("s")
    wid = sub * 2 + core
    i0 = wid * _BPW
    zero16 = lax.broadcasted_iota(jnp.int32, (16,), 0) * 0
    for k in range(_BPW):
        val = zero16 + (i0 + k)
        for v in range(_N // 16):
            buf[pl.ds(k * _N + v * 16, 16)] = val
    pltpu.sync_copy(buf, bv_hbm.at[pl.ds(i0 * _N, _BPW * _N)])


@functools.lru_cache(maxsize=None)
def _sc_bv_call():
    return pl.kernel(
        _sc_bv_body,
        out_type=jax.ShapeDtypeStruct((_B * _N,), jnp.int32),
        mesh=plsc.VectorSubcoreMesh(core_axis_name="c", subcore_axis_name="s"),
        scratch_types=[pltpu.VMEM((_BPW * _N,), jnp.int32)],
    )


def kernel(batch_observations):
    obs = batch_observations
    vel3 = obs[:, 0:2].reshape(_B, 1, 2)
    pos3 = obs[:, 2:4].reshape(_B, 1, 2)
    rel_l = obs[:, 4:4 + 2 * _L].reshape(_B, _L, 2)
    rel_o = obs[:, 4 + 2 * _L:4 + 2 * (_L + _AG - 1)].reshape(_B, _AG - 1, 2)
    comm3 = obs[:, -(_AG - 1):].reshape(_B, _AG - 1, 1)

    ei = _ei_call(jnp.asarray(_TB_NP))
    x3, ea = _tc_call(pos3, vel3, rel_l, rel_o, comm3)
    bv = _sc_bv_call()()

    return (x3.reshape(_B * _N, 8),
            ei,
            ea.reshape(-1),
            bv)
